# Initial kernel scaffold; baseline (speedup 1.0000x reference)
import jax
import jax.numpy as jnp
from jax.experimental import pallas as pl


def kernel(hidden_states, gate_weight, e_score_correction_bias, w_gate_up, w_down, shared_w_gate_up, shared_w_down):
    raise NotImplementedError("write your pallas kernel here")


# speedup vs baseline: 2.0252x; 2.0252x over previous
import functools

import jax
import jax.numpy as jnp
from jax import lax
from jax.experimental import pallas as pl
from jax.experimental.pallas import tpu as pltpu
from jax.experimental.pallas import tpu_sc as plsc

T = 2048
D = 1024
E = 16
K = 2
F = 1024
FS = 1024
C = 384
RSF = 1.0

BT = 256
NBLK = T // BT
DUMP = E * C
NW = 32
TPW = T // NW


def _router_body(x_ref, gw_ref, bias_ref,
                 d0_ref, d1_ref, r0_ref, r1_ref, w0_ref, w1_ref, base_ref):
    pid = pl.program_id(0)

    @pl.when(pid == 0)
    def _():
        base_ref[...] = jnp.zeros_like(base_ref)

    x = x_ref[...]
    gw = gw_ref[...]
    logits = lax.dot_general(x, gw, (((1,), (1,)), ((), ())),
                             preferred_element_type=jnp.float32)
    scores = jax.nn.sigmoid(logits)
    choice = scores + bias_ref[...]

    ie = lax.broadcasted_iota(jnp.int32, (BT, E), 1)
    neg = jnp.float32(-jnp.inf)

    m1 = jnp.max(choice, axis=1, keepdims=True)
    i1 = jnp.min(jnp.where(choice == m1, ie, E), axis=1, keepdims=True)
    oh1 = ie == i1
    choice2 = jnp.where(oh1, neg, choice)
    m2 = jnp.max(choice2, axis=1, keepdims=True)
    i2 = jnp.min(jnp.where(choice2 == m2, ie, E), axis=1, keepdims=True)
    oh2 = ie == i2

    s1 = jnp.sum(jnp.where(oh1, scores, 0.0), axis=1, keepdims=True)
    s2 = jnp.sum(jnp.where(oh2, scores, 0.0), axis=1, keepdims=True)
    denom = s1 + s2 + 1e-20
    w1 = s1 / denom * RSF
    w2 = s2 / denom * RSF

    oh = oh1.astype(jnp.float32) + oh2.astype(jnp.float32)
    ir = lax.broadcasted_iota(jnp.int32, (BT, BT), 0)
    ic = lax.broadcasted_iota(jnp.int32, (BT, BT), 1)
    tri = (ic < ir).astype(jnp.float32)
    prefix = lax.dot_general(tri, oh, (((1,), (0,)), ((), ())),
                             preferred_element_type=jnp.float32)
    base = base_ref[...]
    posmat = base + prefix
    base_ref[...] = base + jnp.sum(oh, axis=0, keepdims=True)

    p1 = jnp.sum(jnp.where(oh1, posmat, 0.0), axis=1, keepdims=True).astype(jnp.int32)
    p2 = jnp.sum(jnp.where(oh2, posmat, 0.0), axis=1, keepdims=True).astype(jnp.int32)

    keep1 = p1 < C
    keep2 = p2 < C
    slot1 = i1 * C + jnp.minimum(p1, C - 1)
    slot2 = i2 * C + jnp.minimum(p2, C - 1)
    d0_ref[...] = jnp.where(keep1, slot1, DUMP)
    d1_ref[...] = jnp.where(keep2, slot2, DUMP)
    r0_ref[...] = slot1
    r1_ref[...] = slot2
    w0_ref[...] = jnp.where(keep1, w1, 0.0)
    w1_ref[...] = jnp.where(keep2, w2, 0.0)


def _router(x, gw, bias2d):
    call = pl.pallas_call(
        _router_body,
        grid=(NBLK,),
        in_specs=[
            pl.BlockSpec((BT, D), lambda i: (i, 0)),
            pl.BlockSpec((E, D), lambda i: (0, 0)),
            pl.BlockSpec((1, E), lambda i: (0, 0)),
        ],
        out_specs=[pl.BlockSpec((BT, 1), lambda i: (i, 0))] * 6,
        out_shape=[jax.ShapeDtypeStruct((T, 1), jnp.int32)] * 4
        + [jax.ShapeDtypeStruct((T, 1), jnp.float32)] * 2,
        scratch_shapes=[pltpu.VMEM((1, E), jnp.float32)],
    )
    return call(x, gw, bias2d)


def _sc_dispatch(x, d0, d1):
    mesh = plsc.VectorSubcoreMesh(core_axis_name="c", subcore_axis_name="s")

    @functools.partial(
        pl.kernel,
        out_type=jax.ShapeDtypeStruct((E * C + 8, D), jnp.float32),
        mesh=mesh,
        scratch_types=[
            pltpu.VMEM((TPW,), jnp.int32),
            pltpu.VMEM((TPW, D), jnp.float32),
            pltpu.SemaphoreType.DMA,
        ],
    )
    def k(x_hbm, d0_hbm, d1_hbm, disp_hbm, idx_v, rows_v, sem):
        wid = lax.axis_index("s") * 2 + lax.axis_index("c")
        base = wid * TPW
        pltpu.sync_copy(x_hbm.at[pl.ds(base, TPW)], rows_v)
        pltpu.sync_copy(d0_hbm.at[pl.ds(base, TPW)], idx_v)
        pltpu.async_copy(rows_v, disp_hbm.at[idx_v], sem).wait()
        pltpu.sync_copy(d1_hbm.at[pl.ds(base, TPW)], idx_v)
        pltpu.async_copy(rows_v, disp_hbm.at[idx_v], sem).wait()

    return k(x, d0, d1)


def _ffn_body(disp_ref, wgu_ref, wdn_ref, eo_ref):
    xb = disp_ref[...].astype(jnp.bfloat16)
    wgu = wgu_ref[0].astype(jnp.bfloat16)
    h = jnp.dot(xb, wgu, preferred_element_type=jnp.float32)
    g = h[:, :F]
    u = h[:, F:]
    act = (g * jax.nn.sigmoid(g) * u).astype(jnp.bfloat16)
    wdn = wdn_ref[0].astype(jnp.bfloat16)
    eo_ref[...] = jnp.dot(act, wdn, preferred_element_type=jnp.float32)


def _ffn(disp, w_gate_up, w_down):
    call = pl.pallas_call(
        _ffn_body,
        grid=(E,),
        in_specs=[
            pl.BlockSpec((C, D), lambda e: (e, 0)),
            pl.BlockSpec((1, D, 2 * F), lambda e: (e, 0, 0)),
            pl.BlockSpec((1, F, D), lambda e: (e, 0, 0)),
        ],
        out_specs=pl.BlockSpec((C, D), lambda e: (e, 0)),
        out_shape=jax.ShapeDtypeStruct((E * C, D), jnp.float32),
    )
    return call(disp, w_gate_up, w_down)


def _sc_combine(eo, r0, r1):
    mesh = plsc.VectorSubcoreMesh(core_axis_name="c", subcore_axis_name="s")

    @functools.partial(
        pl.kernel,
        out_type=[jax.ShapeDtypeStruct((T, D), jnp.float32)] * 2,
        mesh=mesh,
        scratch_types=[
            pltpu.VMEM((TPW,), jnp.int32),
            pltpu.VMEM((TPW, D), jnp.float32),
            pltpu.SemaphoreType.DMA,
        ],
    )
    def k(eo_hbm, r0_hbm, r1_hbm, g0_hbm, g1_hbm, idx_v, rows_v, sem):
        wid = lax.axis_index("s") * 2 + lax.axis_index("c")
        base = wid * TPW
        pltpu.sync_copy(r0_hbm.at[pl.ds(base, TPW)], idx_v)
        pltpu.async_copy(eo_hbm.at[idx_v], rows_v, sem).wait()
        pltpu.sync_copy(rows_v, g0_hbm.at[pl.ds(base, TPW)])
        pltpu.sync_copy(r1_hbm.at[pl.ds(base, TPW)], idx_v)
        pltpu.async_copy(eo_hbm.at[idx_v], rows_v, sem).wait()
        pltpu.sync_copy(rows_v, g1_hbm.at[pl.ds(base, TPW)])

    return k(eo, r0, r1)


def _shared_body(x_ref, wgu_ref, wdn_ref, g0_ref, g1_ref, w0_ref, w1_ref, o_ref):
    xb = x_ref[...].astype(jnp.bfloat16)
    h = jnp.dot(xb, wgu_ref[...].astype(jnp.bfloat16),
                preferred_element_type=jnp.float32)
    g = h[:, :FS]
    u = h[:, FS:]
    act = (g * jax.nn.sigmoid(g) * u).astype(jnp.bfloat16)
    sh = jnp.dot(act, wdn_ref[...].astype(jnp.bfloat16),
                 preferred_element_type=jnp.float32)
    o_ref[...] = sh + w0_ref[...] * g0_ref[...] + w1_ref[...] * g1_ref[...]


def _shared_combine(x, swgu, swdn, g0, g1, w0, w1):
    call = pl.pallas_call(
        _shared_body,
        grid=(NBLK,),
        in_specs=[
            pl.BlockSpec((BT, D), lambda i: (i, 0)),
            pl.BlockSpec((D, 2 * FS), lambda i: (0, 0)),
            pl.BlockSpec((FS, D), lambda i: (0, 0)),
            pl.BlockSpec((BT, D), lambda i: (i, 0)),
            pl.BlockSpec((BT, D), lambda i: (i, 0)),
            pl.BlockSpec((BT, 1), lambda i: (i, 0)),
            pl.BlockSpec((BT, 1), lambda i: (i, 0)),
        ],
        out_specs=pl.BlockSpec((BT, D), lambda i: (i, 0)),
        out_shape=jax.ShapeDtypeStruct((T, D), jnp.float32),
    )
    return call(x, swgu, swdn, g0, g1, w0, w1)


def kernel(hidden_states, gate_weight, e_score_correction_bias,
           w_gate_up, w_down, shared_w_gate_up, shared_w_down):
    x = hidden_states
    bias2d = e_score_correction_bias.reshape(1, E)
    d0, d1, r0, r1, w0, w1 = _router(x, gate_weight, bias2d)
    disp = _sc_dispatch(x, d0.reshape(T), d1.reshape(T))
    eo = _ffn(disp, w_gate_up, w_down)
    g0, g1 = _sc_combine(eo, r0.reshape(T), r1.reshape(T))
    return _shared_combine(x, shared_w_gate_up, shared_w_down, g0, g1, w0, w1)


# speedup vs baseline: 2.2502x; 1.1111x over previous
import functools

import jax
import jax.numpy as jnp
from jax import lax
from jax.experimental import pallas as pl
from jax.experimental.pallas import tpu as pltpu
from jax.experimental.pallas import tpu_sc as plsc

T = 2048
D = 1024
E = 16
K = 2
F = 1024
FS = 1024
C = 384
RSF = 1.0

BT = 256
NBLK = T // BT
DUMP = E * C
NW = 32
TPW = T // NW


def _pack_bf16(xb):
    n2 = xb.shape[1] // 2
    h = lax.bitcast_convert_type(xb[:, :n2], jnp.uint16).astype(jnp.uint32)
    lo = lax.bitcast_convert_type(xb[:, n2:], jnp.uint16).astype(jnp.uint32)
    return lax.bitcast_convert_type((h << 16) | lo, jnp.float32)


def _unpack_bf16(p):
    u = lax.bitcast_convert_type(p, jnp.uint32)
    h = lax.bitcast_convert_type((u >> 16).astype(jnp.uint16), jnp.bfloat16)
    lo = lax.bitcast_convert_type((u & 0xFFFF).astype(jnp.uint16), jnp.bfloat16)
    return jnp.concatenate([h, lo], axis=1)


def _router_body(x_ref, gw_ref, bias_ref,
                 d0_ref, d1_ref, r0_ref, r1_ref, w0_ref, w1_ref, xb_ref,
                 base_ref):
    pid = pl.program_id(0)

    @pl.when(pid == 0)
    def _():
        base_ref[...] = jnp.zeros_like(base_ref)

    x = x_ref[...]
    xb_ref[...] = _pack_bf16(x.astype(jnp.bfloat16))
    gw = gw_ref[...]
    logits = lax.dot_general(x, gw, (((1,), (1,)), ((), ())),
                             preferred_element_type=jnp.float32)
    scores = jax.nn.sigmoid(logits)
    choice = scores + bias_ref[...]

    ie = lax.broadcasted_iota(jnp.int32, (BT, E), 1)
    neg = jnp.float32(-jnp.inf)

    m1 = jnp.max(choice, axis=1, keepdims=True)
    i1 = jnp.min(jnp.where(choice == m1, ie, E), axis=1, keepdims=True)
    oh1 = ie == i1
    choice2 = jnp.where(oh1, neg, choice)
    m2 = jnp.max(choice2, axis=1, keepdims=True)
    i2 = jnp.min(jnp.where(choice2 == m2, ie, E), axis=1, keepdims=True)
    oh2 = ie == i2

    s1 = jnp.sum(jnp.where(oh1, scores, 0.0), axis=1, keepdims=True)
    s2 = jnp.sum(jnp.where(oh2, scores, 0.0), axis=1, keepdims=True)
    denom = s1 + s2 + 1e-20
    w1 = s1 / denom * RSF
    w2 = s2 / denom * RSF

    oh = oh1.astype(jnp.float32) + oh2.astype(jnp.float32)
    ir = lax.broadcasted_iota(jnp.int32, (BT, BT), 0)
    ic = lax.broadcasted_iota(jnp.int32, (BT, BT), 1)
    tri = (ic < ir).astype(jnp.float32)
    prefix = lax.dot_general(tri, oh, (((1,), (0,)), ((), ())),
                             preferred_element_type=jnp.float32)
    base = base_ref[...]
    posmat = base + prefix
    base_ref[...] = base + jnp.sum(oh, axis=0, keepdims=True)

    p1 = jnp.sum(jnp.where(oh1, posmat, 0.0), axis=1, keepdims=True).astype(jnp.int32)
    p2 = jnp.sum(jnp.where(oh2, posmat, 0.0), axis=1, keepdims=True).astype(jnp.int32)

    keep1 = p1 < C
    keep2 = p2 < C
    slot1 = i1 * C + jnp.minimum(p1, C - 1)
    slot2 = i2 * C + jnp.minimum(p2, C - 1)
    d0_ref[...] = jnp.where(keep1, slot1, DUMP)
    d1_ref[...] = jnp.where(keep2, slot2, DUMP)
    r0_ref[...] = slot1
    r1_ref[...] = slot2
    w0_ref[...] = jnp.where(keep1, w1, 0.0)
    w1_ref[...] = jnp.where(keep2, w2, 0.0)


def _router(x, gw, bias2d):
    call = pl.pallas_call(
        _router_body,
        grid=(NBLK,),
        in_specs=[
            pl.BlockSpec((BT, D), lambda i: (i, 0)),
            pl.BlockSpec((E, D), lambda i: (0, 0)),
            pl.BlockSpec((1, E), lambda i: (0, 0)),
        ],
        out_specs=[pl.BlockSpec((BT, 1), lambda i: (i, 0))] * 6
        + [pl.BlockSpec((BT, D // 2), lambda i: (i, 0))],
        out_shape=[jax.ShapeDtypeStruct((T, 1), jnp.int32)] * 4
        + [jax.ShapeDtypeStruct((T, 1), jnp.float32)] * 2
        + [jax.ShapeDtypeStruct((T, D // 2), jnp.float32)],
        scratch_shapes=[pltpu.VMEM((1, E), jnp.float32)],
    )
    return call(x, gw, bias2d)


def _sc_dispatch(x, d0, d1):
    mesh = plsc.VectorSubcoreMesh(core_axis_name="c", subcore_axis_name="s")

    @functools.partial(
        pl.kernel,
        out_type=jax.ShapeDtypeStruct((E * C + 8, D // 2), jnp.float32),
        mesh=mesh,
        scratch_types=[
            pltpu.VMEM((TPW,), jnp.int32),
            pltpu.VMEM((TPW, D // 2), jnp.float32),
            pltpu.SemaphoreType.DMA,
        ],
    )
    def k(x_hbm, d0_hbm, d1_hbm, disp_hbm, idx_v, rows_v, sem):
        wid = lax.axis_index("s") * 2 + lax.axis_index("c")
        base = wid * TPW
        pltpu.sync_copy(x_hbm.at[pl.ds(base, TPW)], rows_v)
        pltpu.sync_copy(d0_hbm.at[pl.ds(base, TPW)], idx_v)
        pltpu.async_copy(rows_v, disp_hbm.at[idx_v], sem).wait()
        pltpu.sync_copy(d1_hbm.at[pl.ds(base, TPW)], idx_v)
        pltpu.async_copy(rows_v, disp_hbm.at[idx_v], sem).wait()

    return k(x, d0, d1)


def _ffn_body(disp_ref, wgu_ref, wdn_ref, eo_ref):
    xb = _unpack_bf16(disp_ref[...])
    wgu = wgu_ref[0].astype(jnp.bfloat16)
    h = jnp.dot(xb, wgu, preferred_element_type=jnp.float32)
    g = h[:, :F]
    u = h[:, F:]
    act = (g * jax.nn.sigmoid(g) * u).astype(jnp.bfloat16)
    wdn = wdn_ref[0].astype(jnp.bfloat16)
    eo = jnp.dot(act, wdn, preferred_element_type=jnp.float32)
    eo_ref[...] = _pack_bf16(eo.astype(jnp.bfloat16))


def _ffn(disp, w_gate_up, w_down):
    call = pl.pallas_call(
        _ffn_body,
        grid=(E,),
        in_specs=[
            pl.BlockSpec((C, D // 2), lambda e: (e, 0)),
            pl.BlockSpec((1, D, 2 * F), lambda e: (e, 0, 0)),
            pl.BlockSpec((1, F, D), lambda e: (e, 0, 0)),
        ],
        out_specs=pl.BlockSpec((C, D // 2), lambda e: (e, 0)),
        out_shape=jax.ShapeDtypeStruct((E * C, D // 2), jnp.float32),
    )
    return call(disp, w_gate_up, w_down)


def _sc_combine(eo, r0, r1):
    mesh = plsc.VectorSubcoreMesh(core_axis_name="c", subcore_axis_name="s")

    @functools.partial(
        pl.kernel,
        out_type=[jax.ShapeDtypeStruct((T, D // 2), jnp.float32)] * 2,
        mesh=mesh,
        scratch_types=[
            pltpu.VMEM((TPW,), jnp.int32),
            pltpu.VMEM((TPW, D // 2), jnp.float32),
            pltpu.SemaphoreType.DMA,
        ],
    )
    def k(eo_hbm, r0_hbm, r1_hbm, g0_hbm, g1_hbm, idx_v, rows_v, sem):
        wid = lax.axis_index("s") * 2 + lax.axis_index("c")
        base = wid * TPW
        pltpu.sync_copy(r0_hbm.at[pl.ds(base, TPW)], idx_v)
        pltpu.async_copy(eo_hbm.at[idx_v], rows_v, sem).wait()
        pltpu.sync_copy(rows_v, g0_hbm.at[pl.ds(base, TPW)])
        pltpu.sync_copy(r1_hbm.at[pl.ds(base, TPW)], idx_v)
        pltpu.async_copy(eo_hbm.at[idx_v], rows_v, sem).wait()
        pltpu.sync_copy(rows_v, g1_hbm.at[pl.ds(base, TPW)])

    return k(eo, r0, r1)


def _shared_body(x_ref, wgu_ref, wdn_ref, g0_ref, g1_ref, w0_ref, w1_ref, o_ref):
    xb = x_ref[...].astype(jnp.bfloat16)
    h = jnp.dot(xb, wgu_ref[...].astype(jnp.bfloat16),
                preferred_element_type=jnp.float32)
    g = h[:, :FS]
    u = h[:, FS:]
    act = (g * jax.nn.sigmoid(g) * u).astype(jnp.bfloat16)
    sh = jnp.dot(act, wdn_ref[...].astype(jnp.bfloat16),
                 preferred_element_type=jnp.float32)
    o_ref[...] = (sh
                  + w0_ref[...] * _unpack_bf16(g0_ref[...]).astype(jnp.float32)
                  + w1_ref[...] * _unpack_bf16(g1_ref[...]).astype(jnp.float32))


def _shared_combine(x, swgu, swdn, g0, g1, w0, w1):
    call = pl.pallas_call(
        _shared_body,
        grid=(NBLK,),
        in_specs=[
            pl.BlockSpec((BT, D), lambda i: (i, 0)),
            pl.BlockSpec((D, 2 * FS), lambda i: (0, 0)),
            pl.BlockSpec((FS, D), lambda i: (0, 0)),
            pl.BlockSpec((BT, D // 2), lambda i: (i, 0)),
            pl.BlockSpec((BT, D // 2), lambda i: (i, 0)),
            pl.BlockSpec((BT, 1), lambda i: (i, 0)),
            pl.BlockSpec((BT, 1), lambda i: (i, 0)),
        ],
        out_specs=pl.BlockSpec((BT, D), lambda i: (i, 0)),
        out_shape=jax.ShapeDtypeStruct((T, D), jnp.float32),
    )
    return call(x, swgu, swdn, g0, g1, w0, w1)


def kernel(hidden_states, gate_weight, e_score_correction_bias,
           w_gate_up, w_down, shared_w_gate_up, shared_w_down):
    x = hidden_states
    bias2d = e_score_correction_bias.reshape(1, E)
    d0, d1, r0, r1, w0, w1, xb16 = _router(x, gate_weight, bias2d)
    disp = _sc_dispatch(xb16, d0.reshape(T), d1.reshape(T))
    eo = _ffn(disp, w_gate_up, w_down)
    g0, g1 = _sc_combine(eo, r0.reshape(T), r1.reshape(T))
    return _shared_combine(x, shared_w_gate_up, shared_w_down, g0, g1, w0, w1)
